# Initial kernel scaffold; baseline (speedup 1.0000x reference)
import functools

import jax
import jax.numpy as jnp
from jax import lax
from jax.experimental import pallas as pl
from jax.experimental.pallas import tpu as pltpu
from jax.experimental.pallas import tpu_sc as plsc

N = 10000
E = 160000
D = 256
DB = 16
DD = 2
H = 256

NC = 2
NS = 16
CH = 128
NCHUNK = E // CH

F_EPS = 1e-5


def _sc_gather_body(h_hbm, row_hbm, col_hbm, src_hbm, tgt_hbm,
                    idx_r, idx_c, buf_r, buf_c, sem_r, sem_c):
    wid = lax.axis_index("s") * NC + lax.axis_index("c")
    nw = NC * NS
    base_trips = NCHUNK // nw
    trips = base_trips + jnp.where(wid < (NCHUNK - base_trips * nw), 1, 0)

    def body(j, carry):
        base = (j * nw + wid) * CH
        pltpu.sync_copy(row_hbm.at[pl.ds(base, CH)], idx_r)
        pltpu.sync_copy(col_hbm.at[pl.ds(base, CH)], idx_c)
        d1 = pltpu.async_copy(h_hbm.at[idx_r], buf_r, sem_r)
        d2 = pltpu.async_copy(h_hbm.at[idx_c], buf_c, sem_c)
        d1.wait()
        d2.wait()
        pltpu.sync_copy(buf_r, src_hbm.at[pl.ds(base, CH)])
        pltpu.sync_copy(buf_c, tgt_hbm.at[pl.ds(base, CH)])
        return carry

    lax.fori_loop(0, trips, body, 0)


_sc_gather = pl.kernel(
    _sc_gather_body,
    out_type=(jax.ShapeDtypeStruct((E, D), jnp.float32),
              jax.ShapeDtypeStruct((E, D), jnp.float32)),
    mesh=plsc.VectorSubcoreMesh(core_axis_name="c", subcore_axis_name="s",
                                num_cores=NC, num_subcores=NS),
    scratch_types=[
        pltpu.VMEM((CH,), jnp.int32),
        pltpu.VMEM((CH,), jnp.int32),
        pltpu.VMEM((CH, D), jnp.float32),
        pltpu.VMEM((CH, D), jnp.float32),
        pltpu.SemaphoreType.DMA,
        pltpu.SemaphoreType.DMA,
    ],
)


HF = D // NC
ROWS_PER_TILE = N // NS


def _sc_scatter_body(mij_hbm, row_hbm, z_hbm, out_hbm, idx_v, mbuf, acc_sp):
    c = lax.axis_index("c")
    s = lax.axis_index("s")
    pltpu.sync_copy(z_hbm, acc_sp.at[pl.ds(s * ROWS_PER_TILE, ROWS_PER_TILE)])
    plsc.subcore_barrier()

    base_trips = NCHUNK // NS
    trips = base_trips + jnp.where(s < (NCHUNK - base_trips * NS), 1, 0)

    def body(j, carry):
        base = (j * NS + s) * CH
        pltpu.sync_copy(row_hbm.at[pl.ds(base, CH)], idx_v)
        pltpu.sync_copy(mij_hbm.at[pl.ds(base, CH), pl.ds(c * HF, HF)], mbuf)
        pltpu.sync_copy(mbuf, acc_sp.at[idx_v], add=True)
        return carry

    lax.fori_loop(0, trips, body, 0)
    plsc.subcore_barrier()
    pltpu.sync_copy(
        acc_sp.at[pl.ds(s * ROWS_PER_TILE, ROWS_PER_TILE)],
        out_hbm.at[pl.ds(s * ROWS_PER_TILE, ROWS_PER_TILE), pl.ds(c * HF, HF)])


_sc_scatter = pl.kernel(
    _sc_scatter_body,
    out_type=jax.ShapeDtypeStruct((N, D), jnp.float32),
    mesh=plsc.VectorSubcoreMesh(core_axis_name="c", subcore_axis_name="s",
                                num_cores=NC, num_subcores=NS),
    scratch_types=[
        pltpu.VMEM((CH,), jnp.int32),
        pltpu.VMEM((CH, HF), jnp.float32),
        pltpu.VMEM_SHARED((N, HF), jnp.float32),
    ],
)


BE = 640
D_E_IN = 2 * D + DB + DD
D_B_IN = DB + H


def _silu(x):
    return x * jax.nn.sigmoid(x)


def _edge_kernel(src, tgt, bnd, dis,
                 w1s, w1t, w1b, w1d, ec1, eg2, ew2, eb2v,
                 wb1b, wb1m, bc1, bg2, bw2, bb2v,
                 mij_ref, bout_ref):
    s = src[...]
    t = tgt[...]
    b = bnd[...]
    d = dis[...]
    s1 = (jnp.sum(s, axis=1, keepdims=True) + jnp.sum(t, axis=1, keepdims=True)
          + jnp.sum(b, axis=1, keepdims=True) + jnp.sum(d, axis=1, keepdims=True))
    q1 = (jnp.sum(s * s, axis=1, keepdims=True) + jnp.sum(t * t, axis=1, keepdims=True)
          + jnp.sum(b * b, axis=1, keepdims=True) + jnp.sum(d * d, axis=1, keepdims=True))
    mean = s1 / D_E_IN
    var = q1 / D_E_IN - mean * mean
    inv = lax.rsqrt(var + F_EPS)
    z = (jnp.dot(s, w1s[...], preferred_element_type=jnp.float32)
         + jnp.dot(t, w1t[...], preferred_element_type=jnp.float32)
         + jnp.dot(b, w1b[...], preferred_element_type=jnp.float32)
         + jnp.dot(d, w1d[...], preferred_element_type=jnp.float32))
    u1 = ec1[0:1, :]
    c1 = ec1[1:2, :]
    a1 = _silu((z - mean * u1) * inv + c1)
    m2 = jnp.mean(a1, axis=1, keepdims=True)
    v2 = jnp.mean(a1 * a1, axis=1, keepdims=True) - m2 * m2
    a2 = (a1 - m2) * lax.rsqrt(v2 + F_EPS) * eg2[0:1, :] + eg2[1:2, :]
    mij = _silu(jnp.dot(a2, ew2[...], preferred_element_type=jnp.float32)
                + eb2v[0:1, :])
    mij_ref[...] = mij
    sb = jnp.sum(b, axis=1, keepdims=True) + jnp.sum(mij, axis=1, keepdims=True)
    qb = (jnp.sum(b * b, axis=1, keepdims=True)
          + jnp.sum(mij * mij, axis=1, keepdims=True))
    meanb = sb / D_B_IN
    varb = qb / D_B_IN - meanb * meanb
    invb = lax.rsqrt(varb + F_EPS)
    zb = (jnp.dot(b, wb1b[...], preferred_element_type=jnp.float32)
          + jnp.dot(mij, wb1m[...], preferred_element_type=jnp.float32))
    ab = _silu((zb - meanb * bc1[0:1, :]) * invb + bc1[1:2, :])
    mb = jnp.mean(ab, axis=1, keepdims=True)
    vb = jnp.mean(ab * ab, axis=1, keepdims=True) - mb * mb
    ab2 = (ab - mb) * lax.rsqrt(vb + F_EPS) * bg2[0:1, :] + bg2[1:2, :]
    bout_ref[...] = _silu(jnp.dot(ab2, bw2[...], preferred_element_type=jnp.float32)
                          + bb2v[0:1, :])


def _tc_edge(src, tgt, bond, dis, weights):
    (w1s, w1t, w1b, w1d, ec1, eg2, ew2, eb2v,
     wb1b, wb1m, bc1, bg2, bw2, bb2v) = weights
    nblk = E // BE
    full = lambda shape: pl.BlockSpec(shape, lambda i: (0, 0))
    return pl.pallas_call(
        _edge_kernel,
        grid=(nblk,),
        in_specs=[
            pl.BlockSpec((BE, D), lambda i: (i, 0)),
            pl.BlockSpec((BE, D), lambda i: (i, 0)),
            pl.BlockSpec((BE, DB), lambda i: (i, 0)),
            pl.BlockSpec((BE, DD), lambda i: (i, 0)),
            full((D, 2 * H)), full((D, 2 * H)), full((DB, 2 * H)),
            full((DD, 2 * H)), full((2, 2 * H)), full((2, 2 * H)),
            full((2 * H, H)), full((1, H)),
            full((DB, H)), full((H, H)), full((2, H)), full((2, H)),
            full((H, DB)), full((1, DB)),
        ],
        out_specs=[
            pl.BlockSpec((BE, H), lambda i: (i, 0)),
            pl.BlockSpec((BE, DB), lambda i: (i, 0)),
        ],
        out_shape=[
            jax.ShapeDtypeStruct((E, H), jnp.float32),
            jax.ShapeDtypeStruct((E, DB), jnp.float32),
        ],
    )(src, tgt, bond, dis, w1s, w1t, w1b, w1d, ec1, eg2, ew2, eb2v,
      wb1b, wb1m, bc1, bg2, bw2, bb2v)


BN = 400


def _node_kernel(h, agg, w1h, w1a, nb1, ng, nw2, nb2, out_ref):
    z = (jnp.dot(h[...], w1h[...], preferred_element_type=jnp.float32)
         + jnp.dot(agg[...], w1a[...], preferred_element_type=jnp.float32)
         + nb1[0:1, :])
    a = _silu(z)
    m = jnp.mean(a, axis=1, keepdims=True)
    v = jnp.mean(a * a, axis=1, keepdims=True) - m * m
    a2 = (a - m) * lax.rsqrt(v + F_EPS) * ng[0:1, :] + ng[1:2, :]
    out_ref[...] = (jnp.dot(a2, nw2[...], preferred_element_type=jnp.float32)
                    + nb2[0:1, :])


def _tc_node(h, agg, weights):
    w1h, w1a, nb1, ng, nw2, nb2 = weights
    nblk = N // BN
    full = lambda shape: pl.BlockSpec(shape, lambda i: (0, 0))
    return pl.pallas_call(
        _node_kernel,
        grid=(nblk,),
        in_specs=[
            pl.BlockSpec((BN, D), lambda i: (i, 0)),
            pl.BlockSpec((BN, D), lambda i: (i, 0)),
            full((D, H)), full((D, H)), full((1, H)), full((2, H)),
            full((H, D)), full((1, D)),
        ],
        out_specs=pl.BlockSpec((BN, D), lambda i: (i, 0)),
        out_shape=jax.ShapeDtypeStruct((N, D), jnp.float32),
    )(h, agg, w1h, w1a, nb1, ng, nw2, nb2)


def kernel(dis_emb, h, edge_index, bond, params):
    p = params
    row = edge_index[0].astype(jnp.int32)
    col = edge_index[1].astype(jnp.int32)

    g1 = p['e_ln1_g'][:, None]
    w1 = p['e_W1'] * g1
    u1 = jnp.sum(w1, axis=0, keepdims=True)
    c1 = p['e_ln1_b'][None, :] @ p['e_W1'] + p['e_b1'][None, :]
    ec1 = jnp.concatenate([u1, c1], axis=0)
    eg2 = jnp.stack([p['e_ln2_g'], p['e_ln2_b']], axis=0)
    eb2v = p['e_b2'][None, :]

    wb1 = p['b_W1'] * p['b_ln1_g'][:, None]
    ub = jnp.sum(wb1, axis=0, keepdims=True)
    cb = p['b_ln1_b'][None, :] @ p['b_W1'] + p['b_b1'][None, :]
    bc1 = jnp.concatenate([ub, cb], axis=0)
    bg2 = jnp.stack([p['b_ln2_g'], p['b_ln2_b']], axis=0)
    bb2v = p['b_b2'][None, :]

    edge_weights = (
        w1[:D], w1[D:2 * D], w1[2 * D:2 * D + DB], w1[2 * D + DB:],
        ec1, eg2, p['e_W2'], eb2v,
        wb1[:DB], wb1[DB:], bc1, bg2, p['b_W2'], bb2v,
    )
    node_weights = (
        p['n_W1'][:D], p['n_W1'][D:], p['n_b1'][None, :],
        jnp.stack([p['n_ln_g'], p['n_ln_b']], axis=0),
        p['n_W2'], p['n_b2'][None, :],
    )

    src, tgt = _sc_gather(h, row, col)
    mij, bond_out = _tc_edge(src, tgt, bond, dis_emb, edge_weights)
    zrows = jnp.zeros((ROWS_PER_TILE, HF), jnp.float32)
    agg = _sc_scatter(mij, row, zrows)
    h_out = _tc_node(h, agg, node_weights)
    return (h_out, bond_out)


# speedup vs baseline: 2.2932x; 2.2932x over previous
import functools

import jax
import jax.numpy as jnp
from jax import lax
from jax.experimental import pallas as pl
from jax.experimental.pallas import tpu as pltpu
from jax.experimental.pallas import tpu_sc as plsc

N = 10000
E = 160000
D = 256
DB = 16
DD = 2
H = 256

NC = 2
NS = 16
CH = 128
NCHUNK = E // CH

F_EPS = 1e-5


def _sc_gather_body(h_hbm, row_hbm, col_hbm, src_hbm, tgt_hbm,
                    idx_r, idx_c, buf_r, buf_c, sem_r, sem_c):
    wid = lax.axis_index("s") * NC + lax.axis_index("c")
    nw = NC * NS
    base_trips = NCHUNK // nw
    trips = base_trips + jnp.where(wid < (NCHUNK - base_trips * nw), 1, 0)

    def body(j, carry):
        base = (j * nw + wid) * CH
        pltpu.sync_copy(row_hbm.at[pl.ds(base, CH)], idx_r)
        pltpu.sync_copy(col_hbm.at[pl.ds(base, CH)], idx_c)
        d1 = pltpu.async_copy(h_hbm.at[idx_r], buf_r, sem_r)
        d2 = pltpu.async_copy(h_hbm.at[idx_c], buf_c, sem_c)
        d1.wait()
        d2.wait()
        pltpu.sync_copy(buf_r, src_hbm.at[pl.ds(base, CH)])
        pltpu.sync_copy(buf_c, tgt_hbm.at[pl.ds(base, CH)])
        return carry

    lax.fori_loop(0, trips, body, 0)


@functools.cache
def _sc_gather():
    return pl.kernel(
        _sc_gather_body,
        out_type=(jax.ShapeDtypeStruct((E, D), jnp.float32),
                  jax.ShapeDtypeStruct((E, D), jnp.float32)),
        mesh=plsc.VectorSubcoreMesh(core_axis_name="c", subcore_axis_name="s",
                                    num_cores=NC, num_subcores=NS),
        scratch_types=[
            pltpu.VMEM((CH,), jnp.int32),
            pltpu.VMEM((CH,), jnp.int32),
            pltpu.VMEM((CH, D), jnp.float32),
            pltpu.VMEM((CH, D), jnp.float32),
            pltpu.SemaphoreType.DMA,
            pltpu.SemaphoreType.DMA,
        ],
    )


HF = D // NC
STRIPE = 624
STRIPE_LAST = N - (NS - 1) * STRIPE


def _sc_scatter_body(mij_hbm, row_hbm, z_hbm, out_hbm, idx_v, mbuf, acc_sp):
    c = lax.axis_index("c")
    s = lax.axis_index("s")
    slot = s * STRIPE

    @pl.when(s < NS - 1)
    def _():
        pltpu.sync_copy(z_hbm.at[pl.ds(0, STRIPE)],
                        acc_sp.at[pl.ds(slot, STRIPE)])

    @pl.when(s == NS - 1)
    def _():
        pltpu.sync_copy(z_hbm, acc_sp.at[pl.ds(slot, STRIPE_LAST)])

    plsc.subcore_barrier()

    base_trips = NCHUNK // NS
    trips = base_trips + jnp.where(s < (NCHUNK - base_trips * NS), 1, 0)

    def body(j, carry):
        base = (j * NS + s) * CH
        pltpu.sync_copy(row_hbm.at[pl.ds(base, CH)], idx_v)
        pltpu.sync_copy(mij_hbm.at[pl.ds(base, CH), pl.ds(c * HF, HF)], mbuf)
        pltpu.sync_copy(mbuf, acc_sp.at[idx_v], add=True)
        return carry

    lax.fori_loop(0, trips, body, 0)
    plsc.subcore_barrier()

    @pl.when(s < NS - 1)
    def _():
        pltpu.sync_copy(acc_sp.at[pl.ds(slot, STRIPE)],
                        out_hbm.at[pl.ds(slot, STRIPE), pl.ds(c * HF, HF)])

    @pl.when(s == NS - 1)
    def _():
        pltpu.sync_copy(acc_sp.at[pl.ds(slot, STRIPE_LAST)],
                        out_hbm.at[pl.ds(slot, STRIPE_LAST), pl.ds(c * HF, HF)])


@functools.cache
def _sc_scatter():
    return pl.kernel(
        _sc_scatter_body,
        out_type=jax.ShapeDtypeStruct((N, D), jnp.float32),
        mesh=plsc.VectorSubcoreMesh(core_axis_name="c", subcore_axis_name="s",
                                    num_cores=NC, num_subcores=NS),
        scratch_types=[
            pltpu.VMEM((CH,), jnp.int32),
            pltpu.VMEM((CH, HF), jnp.float32),
            pltpu.VMEM_SHARED((N, HF), jnp.float32),
        ],
    )


BE = 640
D_E_IN = 2 * D + DB + DD
D_B_IN = DB + H


def _silu(x):
    return x * jax.nn.sigmoid(x)


def _edge_kernel(src, tgt, bnd, dis,
                 w1s, w1t, w1b, w1d, ec1, eg2, ew2, eb2v,
                 wb1b, wb1m, bc1, bg2, bw2, bb2v,
                 mij_ref, bout_ref):
    s = src[...]
    t = tgt[...]
    b = bnd[...]
    d = dis[...]
    s1 = (jnp.sum(s, axis=1, keepdims=True) + jnp.sum(t, axis=1, keepdims=True)
          + jnp.sum(b, axis=1, keepdims=True) + jnp.sum(d, axis=1, keepdims=True))
    q1 = (jnp.sum(s * s, axis=1, keepdims=True) + jnp.sum(t * t, axis=1, keepdims=True)
          + jnp.sum(b * b, axis=1, keepdims=True) + jnp.sum(d * d, axis=1, keepdims=True))
    mean = s1 / D_E_IN
    var = q1 / D_E_IN - mean * mean
    inv = lax.rsqrt(var + F_EPS)
    z = (jnp.dot(s, w1s[...], preferred_element_type=jnp.float32)
         + jnp.dot(t, w1t[...], preferred_element_type=jnp.float32)
         + jnp.dot(b, w1b[...], preferred_element_type=jnp.float32)
         + jnp.dot(d, w1d[...], preferred_element_type=jnp.float32))
    u1 = ec1[0:1, :]
    c1 = ec1[1:2, :]
    a1 = _silu((z - mean * u1) * inv + c1)
    m2 = jnp.mean(a1, axis=1, keepdims=True)
    v2 = jnp.mean(a1 * a1, axis=1, keepdims=True) - m2 * m2
    a2 = (a1 - m2) * lax.rsqrt(v2 + F_EPS) * eg2[0:1, :] + eg2[1:2, :]
    mij = _silu(jnp.dot(a2, ew2[...], preferred_element_type=jnp.float32)
                + eb2v[0:1, :])
    mij_ref[...] = mij
    sb = jnp.sum(b, axis=1, keepdims=True) + jnp.sum(mij, axis=1, keepdims=True)
    qb = (jnp.sum(b * b, axis=1, keepdims=True)
          + jnp.sum(mij * mij, axis=1, keepdims=True))
    meanb = sb / D_B_IN
    varb = qb / D_B_IN - meanb * meanb
    invb = lax.rsqrt(varb + F_EPS)
    zb = (jnp.dot(b, wb1b[...], preferred_element_type=jnp.float32)
          + jnp.dot(mij, wb1m[...], preferred_element_type=jnp.float32))
    ab = _silu((zb - meanb * bc1[0:1, :]) * invb + bc1[1:2, :])
    mb = jnp.mean(ab, axis=1, keepdims=True)
    vb = jnp.mean(ab * ab, axis=1, keepdims=True) - mb * mb
    ab2 = (ab - mb) * lax.rsqrt(vb + F_EPS) * bg2[0:1, :] + bg2[1:2, :]
    bout_ref[...] = _silu(jnp.dot(ab2, bw2[...], preferred_element_type=jnp.float32)
                          + bb2v[0:1, :])


def _tc_edge(src, tgt, bond, dis, weights):
    (w1s, w1t, w1b, w1d, ec1, eg2, ew2, eb2v,
     wb1b, wb1m, bc1, bg2, bw2, bb2v) = weights
    nblk = E // BE
    full = lambda shape: pl.BlockSpec(shape, lambda i: (0, 0))
    return pl.pallas_call(
        _edge_kernel,
        grid=(nblk,),
        in_specs=[
            pl.BlockSpec((BE, D), lambda i: (i, 0)),
            pl.BlockSpec((BE, D), lambda i: (i, 0)),
            pl.BlockSpec((BE, DB), lambda i: (i, 0)),
            pl.BlockSpec((BE, DD), lambda i: (i, 0)),
            full((D, 2 * H)), full((D, 2 * H)), full((DB, 2 * H)),
            full((DD, 2 * H)), full((2, 2 * H)), full((2, 2 * H)),
            full((2 * H, H)), full((1, H)),
            full((DB, H)), full((H, H)), full((2, H)), full((2, H)),
            full((H, DB)), full((1, DB)),
        ],
        out_specs=[
            pl.BlockSpec((BE, H), lambda i: (i, 0)),
            pl.BlockSpec((BE, DB), lambda i: (i, 0)),
        ],
        out_shape=[
            jax.ShapeDtypeStruct((E, H), jnp.float32),
            jax.ShapeDtypeStruct((E, DB), jnp.float32),
        ],
    )(src, tgt, bond, dis, w1s, w1t, w1b, w1d, ec1, eg2, ew2, eb2v,
      wb1b, wb1m, bc1, bg2, bw2, bb2v)


BN = 400


def _node_kernel(h, agg, w1h, w1a, nb1, ng, nw2, nb2, out_ref):
    z = (jnp.dot(h[...], w1h[...], preferred_element_type=jnp.float32)
         + jnp.dot(agg[...], w1a[...], preferred_element_type=jnp.float32)
         + nb1[0:1, :])
    a = _silu(z)
    m = jnp.mean(a, axis=1, keepdims=True)
    v = jnp.mean(a * a, axis=1, keepdims=True) - m * m
    a2 = (a - m) * lax.rsqrt(v + F_EPS) * ng[0:1, :] + ng[1:2, :]
    out_ref[...] = (jnp.dot(a2, nw2[...], preferred_element_type=jnp.float32)
                    + nb2[0:1, :])


def _tc_node(h, agg, weights):
    w1h, w1a, nb1, ng, nw2, nb2 = weights
    nblk = N // BN
    full = lambda shape: pl.BlockSpec(shape, lambda i: (0, 0))
    return pl.pallas_call(
        _node_kernel,
        grid=(nblk,),
        in_specs=[
            pl.BlockSpec((BN, D), lambda i: (i, 0)),
            pl.BlockSpec((BN, D), lambda i: (i, 0)),
            full((D, H)), full((D, H)), full((1, H)), full((2, H)),
            full((H, D)), full((1, D)),
        ],
        out_specs=pl.BlockSpec((BN, D), lambda i: (i, 0)),
        out_shape=jax.ShapeDtypeStruct((N, D), jnp.float32),
    )(h, agg, w1h, w1a, nb1, ng, nw2, nb2)


def _prep_weights(p):
    g1 = p['e_ln1_g'][:, None]
    w1 = p['e_W1'] * g1
    u1 = jnp.sum(w1, axis=0, keepdims=True)
    c1 = p['e_ln1_b'][None, :] @ p['e_W1'] + p['e_b1'][None, :]
    ec1 = jnp.concatenate([u1, c1], axis=0)
    eg2 = jnp.stack([p['e_ln2_g'], p['e_ln2_b']], axis=0)
    eb2v = p['e_b2'][None, :]

    wb1 = p['b_W1'] * p['b_ln1_g'][:, None]
    ub = jnp.sum(wb1, axis=0, keepdims=True)
    cb = p['b_ln1_b'][None, :] @ p['b_W1'] + p['b_b1'][None, :]
    bc1 = jnp.concatenate([ub, cb], axis=0)
    bg2 = jnp.stack([p['b_ln2_g'], p['b_ln2_b']], axis=0)
    bb2v = p['b_b2'][None, :]

    edge_weights = (
        w1[:D], w1[D:2 * D], w1[2 * D:2 * D + DB], w1[2 * D + DB:],
        ec1, eg2, p['e_W2'], eb2v,
        wb1[:DB], wb1[DB:], bc1, bg2, p['b_W2'], bb2v,
    )
    node_weights = (
        p['n_W1'][:D], p['n_W1'][D:], p['n_b1'][None, :],
        jnp.stack([p['n_ln_g'], p['n_ln_b']], axis=0),
        p['n_W2'], p['n_b2'][None, :],
    )
    return edge_weights, node_weights


def kernel(dis_emb, h, edge_index, bond, params):
    row = edge_index[0].astype(jnp.int32)
    col = edge_index[1].astype(jnp.int32)
    edge_weights, node_weights = _prep_weights(params)

    src, tgt = _sc_gather()(h, row, col)
    mij, bond_out = _tc_edge(src, tgt, bond, dis_emb, edge_weights)
    zrows = jnp.zeros((STRIPE_LAST, HF), jnp.float32)
    agg = _sc_scatter()(mij, row, zrows)
    h_out = _tc_node(h, agg, node_weights)
    return (h_out, bond_out)
